# Initial kernel scaffold; baseline (speedup 1.0000x reference)
import functools

import jax
import jax.numpy as jnp
from jax import lax
from jax.experimental import pallas as pl
from jax.experimental.pallas import tpu as pltpu
from jax.experimental.pallas import tpu_sc as plsc

N_REL = 100
ARITY = 32
DIM = 128
BATCH = 16384
PAIRS = ARITY * (ARITY - 1)

_IDX_CHUNK = 128


@functools.cache
def _build():
    info = plsc.get_sparse_core_info()
    nc, ns = info.num_cores, info.num_subcores
    nw = nc * ns
    b_per_w = BATCH // nw
    n_chunk = b_per_w // _IDX_CHUNK
    n_grp = b_per_w // 16
    mesh = plsc.VectorSubcoreMesh(core_axis_name="c", subcore_axis_name="s")

    @functools.partial(
        pl.kernel,
        mesh=mesh,
        out_type=jax.ShapeDtypeStruct((BATCH, DIM), jnp.float32),
        scratch_types=[
            pltpu.VMEM((b_per_w * 3,), jnp.int32),
            pltpu.VMEM((n_chunk, _IDX_CHUNK), jnp.int32),
            pltpu.VMEM((b_per_w, DIM), jnp.float32),
            pltpu.SemaphoreType.DMA,
        ],
    )
    def sc_gather(br_hbm, table_hbm, out_hbm, br_v, idx_v, rows_v, sem):
        wid = lax.axis_index("s") * nc + lax.axis_index("c")
        base = wid * b_per_w
        pltpu.sync_copy(br_hbm.at[pl.ds(base * 3, b_per_w * 3)], br_v)
        lane3 = lax.iota(jnp.int32, 16) * 3
        for g in range(n_grp):
            pos = lane3 + (g * 48)
            i_r = plsc.load_gather(br_v, [pos])
            i1 = plsc.load_gather(br_v, [pos + 1])
            i2 = plsc.load_gather(br_v, [pos + 2])
            i2_adj = jnp.where(i2 > i1, i2 - 1, i2)
            flat = i_r * PAIRS + i1 * (ARITY - 1) + i2_adj
            idx_v[g // 8, pl.ds((g % 8) * 16, 16)] = flat
        copies = [
            pltpu.async_copy(
                table_hbm.at[idx_v.at[j]],
                rows_v.at[pl.ds(j * _IDX_CHUNK, _IDX_CHUNK)],
                sem,
            )
            for j in range(n_chunk)
        ]
        for c in copies:
            c.wait()
        pltpu.sync_copy(rows_v, out_hbm.at[pl.ds(base, b_per_w)])

    return sc_gather


def kernel(batch_rels, table):
    return _build()(batch_rels.reshape(-1), table)


# speedup vs baseline: 1.5528x; 1.5528x over previous
import functools

import jax
import jax.numpy as jnp
from jax import lax
from jax.experimental import pallas as pl
from jax.experimental.pallas import tpu as pltpu
from jax.experimental.pallas import tpu_sc as plsc

N_REL = 100
ARITY = 32
DIM = 128
BATCH = 16384
PAIRS = ARITY * (ARITY - 1)

_IDX_CHUNK = 128


@functools.cache
def _build():
    info = plsc.get_sparse_core_info()
    nc, ns = info.num_cores, info.num_subcores
    nw = nc * ns
    b_per_w = BATCH // nw
    n_chunk = b_per_w // _IDX_CHUNK
    n_grp = b_per_w // 16
    mesh = plsc.VectorSubcoreMesh(core_axis_name="c", subcore_axis_name="s")

    @functools.partial(
        pl.kernel,
        mesh=mesh,
        out_type=jax.ShapeDtypeStruct((BATCH, DIM), jnp.float32),
        scratch_types=[
            pltpu.VMEM((3, b_per_w), jnp.int32),
            pltpu.VMEM((n_chunk, _IDX_CHUNK), jnp.int32),
            pltpu.VMEM((b_per_w, DIM), jnp.float32),
            pltpu.SemaphoreType.DMA,
        ],
    )
    def sc_gather(br_hbm, table_hbm, out_hbm, br_v, idx_v, rows_v, sem):
        wid = lax.axis_index("s") * nc + lax.axis_index("c")
        base = wid * b_per_w
        pltpu.sync_copy(br_hbm.at[:, pl.ds(base, b_per_w)], br_v)
        for g in range(n_grp):
            i_r = br_v[0, pl.ds(g * 16, 16)]
            i1 = br_v[1, pl.ds(g * 16, 16)]
            i2 = br_v[2, pl.ds(g * 16, 16)]
            i2_adj = jnp.where(i2 > i1, i2 - 1, i2)
            flat = i_r * PAIRS + i1 * (ARITY - 1) + i2_adj
            idx_v[g // 8, pl.ds((g % 8) * 16, 16)] = flat
        copies = [
            pltpu.async_copy(
                table_hbm.at[idx_v.at[j]],
                rows_v.at[pl.ds(j * _IDX_CHUNK, _IDX_CHUNK)],
                sem,
            )
            for j in range(n_chunk)
        ]
        for c in copies:
            c.wait()
        pltpu.sync_copy(rows_v, out_hbm.at[pl.ds(base, b_per_w)])

    return sc_gather


def kernel(batch_rels, table):
    return _build()(batch_rels.T, table)
